# baseline (speedup 1.0000x reference)
import jax
import jax.numpy as jnp
from jax.experimental import pallas as pl
from jax.experimental.pallas import tpu as pltpu

_XH = 16
_ZH = 8


_XCH = 32
_ZCH = 8


def _chunks(n, head, k):
    total = n - head
    per = ((total // k) // 8) * 8
    out = []
    start = head
    for i in range(k):
        size = per if i < k - 1 else n - start
        out.append((start, size))
        start += size
    return out


def _body(x_hbm, y_ref, z_hbm, w_ref, xo_hbm, zo_hbm, xt, zt,
          sx, sz, sxi, szi, sxo, szo):
    nx = x_hbm.shape[0]
    nz = z_hbm.shape[0]
    big = []
    for j, (s, sz_rows) in enumerate(_chunks(nx, _XH, _XCH)):
        c = pltpu.make_async_copy(x_hbm.at[pl.ds(s, sz_rows), :],
                                  xo_hbm.at[pl.ds(s, sz_rows), :], sx.at[j])
        c.start()
        big.append(c)
    for j, (s, sz_rows) in enumerate(_chunks(nz, _ZH, _ZCH)):
        c = pltpu.make_async_copy(z_hbm.at[pl.ds(s, sz_rows), :],
                                  zo_hbm.at[pl.ds(s, sz_rows), :], sz.at[j])
        c.start()
        big.append(c)
    cxi = pltpu.make_async_copy(x_hbm.at[pl.ds(0, _XH), :], xt, sxi)
    cxi.start()
    czi = pltpu.make_async_copy(z_hbm.at[pl.ds(0, _ZH), :], zt, szi)
    czi.start()

    cxi.wait()
    r = jax.lax.broadcasted_iota(jnp.int32, (_XH, 256), 0)
    b = xt[...]
    b = jnp.where(r == 10, y_ref[0, :][None, :], b)
    b = jnp.where(r == 2, y_ref[1, :][None, :], b)
    b = jnp.where(r == 1, jnp.float32(45.0), b)
    xt[...] = b

    czi.wait()
    rz = jax.lax.broadcasted_iota(jnp.int32, (_ZH, 1024), 0)
    cz = jax.lax.broadcasted_iota(jnp.int32, (_ZH, 1024), 1)
    add = (w_ref[0] * ((rz == 1) & (cz == 3)).astype(jnp.float32)
           + w_ref[1] * ((rz == 0) & (cz == 2)).astype(jnp.float32)
           + w_ref[2] * ((rz == 0) & (cz == 1)).astype(jnp.float32))
    zt[...] = zt[...] + add

    cxo = pltpu.make_async_copy(xt, xo_hbm.at[pl.ds(0, _XH), :], sxo)
    cxo.start()
    czo = pltpu.make_async_copy(zt, zo_hbm.at[pl.ds(0, _ZH), :], szo)
    czo.start()
    cxo.wait()
    czo.wait()
    for c in big:
        c.wait()


def kernel(x, y, z, w):
    xo, zo = pl.pallas_call(
        _body,
        in_specs=[
            pl.BlockSpec(memory_space=pl.ANY),
            pl.BlockSpec(memory_space=pltpu.VMEM),
            pl.BlockSpec(memory_space=pl.ANY),
            pl.BlockSpec(memory_space=pltpu.SMEM),
        ],
        out_specs=[
            pl.BlockSpec(memory_space=pl.ANY),
            pl.BlockSpec(memory_space=pl.ANY),
        ],
        out_shape=[
            jax.ShapeDtypeStruct(x.shape, x.dtype),
            jax.ShapeDtypeStruct(z.shape, z.dtype),
        ],
        scratch_shapes=[
            pltpu.VMEM((_XH, 256), jnp.float32),
            pltpu.VMEM((_ZH, 1024), jnp.float32),
            pltpu.SemaphoreType.DMA((_XCH,)),
            pltpu.SemaphoreType.DMA((_ZCH,)),
            pltpu.SemaphoreType.DMA,
            pltpu.SemaphoreType.DMA,
            pltpu.SemaphoreType.DMA,
            pltpu.SemaphoreType.DMA,
        ],
    )(x, y, z, w)
    return (xo, zo)


# speedup vs baseline: 46.0494x; 46.0494x over previous
import jax
import jax.numpy as jnp
from jax.experimental import pallas as pl
from jax.experimental.pallas import tpu as pltpu

_G = 128
_XR = 262144 // _G
_ZR = 16384 // _G


def _body(y_ref, w_ref, x_ref, z_ref, xo_ref, zo_ref):
    i = pl.program_id(0)

    @pl.when(i == 0)
    def _patch():
        r = jax.lax.broadcasted_iota(jnp.int32, (_XR, 256), 0)
        b = x_ref[...]
        b = jnp.where(r == 10, y_ref[0, :][None, :], b)
        b = jnp.where(r == 2, y_ref[1, :][None, :], b)
        b = jnp.where(r == 1, jnp.float32(45.0), b)
        xo_ref[...] = b
        rz = jax.lax.broadcasted_iota(jnp.int32, (_ZR, 1024), 0)
        cz = jax.lax.broadcasted_iota(jnp.int32, (_ZR, 1024), 1)
        add = (w_ref[0] * ((rz == 1) & (cz == 3)).astype(jnp.float32)
               + w_ref[1] * ((rz == 0) & (cz == 2)).astype(jnp.float32)
               + w_ref[2] * ((rz == 0) & (cz == 1)).astype(jnp.float32))
        zo_ref[...] = z_ref[...] + add

    @pl.when(i != 0)
    def _copy():
        xo_ref[...] = x_ref[...]
        zo_ref[...] = z_ref[...]


def kernel(x, y, z, w):
    xo, zo = pl.pallas_call(
        _body,
        grid=(_G,),
        in_specs=[
            pl.BlockSpec((2, 256), lambda i: (0, 0)),
            pl.BlockSpec(memory_space=pltpu.SMEM),
            pl.BlockSpec((_XR, 256), lambda i: (i, 0)),
            pl.BlockSpec((_ZR, 1024), lambda i: (i, 0)),
        ],
        out_specs=[
            pl.BlockSpec((_XR, 256), lambda i: (i, 0)),
            pl.BlockSpec((_ZR, 1024), lambda i: (i, 0)),
        ],
        out_shape=[
            jax.ShapeDtypeStruct(x.shape, x.dtype),
            jax.ShapeDtypeStruct(z.shape, z.dtype),
        ],
        compiler_params=pltpu.CompilerParams(
            dimension_semantics=("arbitrary",)),
    )(y, w, x, z)
    return (xo, zo)
